# speedup vs baseline: 3.5978x; 3.5978x over previous
import functools

import jax
import jax.numpy as jnp
from jax import lax
from jax.experimental import pallas as pl
from jax.experimental.pallas import tpu as pltpu
from jax.experimental.pallas import tpu_sc as plsc

MAXLEN = 200
EMBED = 128
BATCH = 1024
NW = 32
BPW = BATCH // NW
HALF = MAXLEN // 2
LANES = 16


def _body(x_hbm, tok_hbm, pos_hbm, out_hbm, pos_v, idx_v, rows_v, sem):
    wid = lax.axis_index("s") * 2 + lax.axis_index("c")
    pltpu.sync_copy(pos_hbm, pos_v)

    @pl.loop(0, BPW)
    def _chunk(i):
        b = wid * BPW + i
        pltpu.sync_copy(x_hbm.at[b], idx_v)
        pltpu.async_copy(
            tok_hbm.at[idx_v.at[0]], rows_v.at[pl.ds(0, HALF)], sem
        ).wait()
        pltpu.async_copy(
            tok_hbm.at[idx_v.at[1]], rows_v.at[pl.ds(HALF, HALF)], sem
        ).wait()

        @pl.loop(0, MAXLEN)
        def _row(r):
            for c in range(EMBED // LANES):
                sl = pl.ds(c * LANES, LANES)
                plsc.addupdate(rows_v.at[r, sl], pos_v[r, sl])

        pltpu.sync_copy(rows_v, out_hbm.at[b])


def kernel(x, token_table, pos_table):
    x3 = x.reshape(BATCH, 2, HALF).astype(jnp.int32)
    mesh = plsc.VectorSubcoreMesh(core_axis_name="c", subcore_axis_name="s")
    f = pl.kernel(
        _body,
        out_type=jax.ShapeDtypeStruct((BATCH, MAXLEN, EMBED), jnp.float32),
        mesh=mesh,
        scratch_types=[
            pltpu.VMEM((MAXLEN, EMBED), jnp.float32),
            pltpu.VMEM((2, HALF), jnp.int32),
            pltpu.VMEM((MAXLEN, EMBED), jnp.float32),
            pltpu.SemaphoreType.DMA,
        ],
    )
    return f(x3, token_table, pos_table)


# speedup vs baseline: 6.1313x; 1.7042x over previous
import jax
import jax.numpy as jnp
from jax import lax
from jax.experimental import pallas as pl
from jax.experimental.pallas import tpu as pltpu
from jax.experimental.pallas import tpu_sc as plsc

MAXLEN = 200
EMBED = 128
BATCH = 1024
NW = 32
BPW = BATCH // NW
HALF = MAXLEN // 2
LANES = 16


def _body(x_hbm, tok_hbm, pos_hbm, out_hbm, pos_v, idx_v, rows0, rows1,
          sg0, sg1, so0, so1):
    wid = lax.axis_index("s") * 2 + lax.axis_index("c")
    pltpu.sync_copy(pos_hbm, pos_v)
    pltpu.sync_copy(x_hbm.at[wid], idx_v)

    rows = (rows0, rows1)
    sg = (sg0, sg1)
    so = (so0, so1)
    store_desc = [None, None]

    def start_gather(i):
        b = i % 2
        return (
            pltpu.async_copy(
                tok_hbm.at[idx_v.at[2 * i]], rows[b].at[pl.ds(0, HALF)], sg[b]
            ),
            pltpu.async_copy(
                tok_hbm.at[idx_v.at[2 * i + 1]],
                rows[b].at[pl.ds(HALF, HALF)],
                sg[b],
            ),
        )

    gather_desc = start_gather(0)
    for i in range(BPW):
        b = i % 2
        pending = gather_desc
        if i + 1 < BPW:
            nb = (i + 1) % 2
            if store_desc[nb] is not None:
                store_desc[nb].wait()
                store_desc[nb] = None
            gather_desc = start_gather(i + 1)
        pending[0].wait()
        pending[1].wait()

        @pl.loop(0, MAXLEN, unroll=2)
        def _row(r):
            for c in range(EMBED // LANES):
                sl = pl.ds(c * LANES, LANES)
                plsc.addupdate(rows[b].at[r, sl], pos_v[r, sl])

        store_desc[b] = pltpu.async_copy(rows[b], out_hbm.at[wid * BPW + i], so[b])

    for d in store_desc:
        if d is not None:
            d.wait()


def kernel(x, token_table, pos_table):
    x3 = x.reshape(NW, 2 * BPW, HALF).astype(jnp.int32)
    mesh = plsc.VectorSubcoreMesh(core_axis_name="c", subcore_axis_name="s")
    f = pl.kernel(
        _body,
        out_type=jax.ShapeDtypeStruct((BATCH, MAXLEN, EMBED), jnp.float32),
        mesh=mesh,
        scratch_types=[
            pltpu.VMEM((MAXLEN, EMBED), jnp.float32),
            pltpu.VMEM((2 * BPW, HALF), jnp.int32),
            pltpu.VMEM((MAXLEN, EMBED), jnp.float32),
            pltpu.VMEM((MAXLEN, EMBED), jnp.float32),
            pltpu.SemaphoreType.DMA,
            pltpu.SemaphoreType.DMA,
            pltpu.SemaphoreType.DMA,
            pltpu.SemaphoreType.DMA,
        ],
    )
    return f(x3, token_table, pos_table)
